# baseline (speedup 1.0000x reference)
import functools

import jax
import jax.numpy as jnp
from jax import lax
from jax.experimental import pallas as pl
from jax.experimental.pallas import tpu as pltpu
from jax.experimental.pallas import tpu_sc as plsc

N_NODES = 10000
N_EDGES = 320000
IN_CH = 128
HID = 256

NC = 2
NS = 16
LANES = 16

CHUNK = 24
SUPER = 4
CHUNKS_PER_TILE = 856
N_SUPER = CHUNKS_PER_TILE // SUPER
TILE_EDGES = CHUNKS_PER_TILE * CHUNK
E_PAD = NS * TILE_EDGES
W_EDGES = E_PAD // (NC * NS)
HIST_W = 10240
HALF = N_NODES // 2
ACC_R = 10112
SLAB = ACC_R // NS
TSLAB = 320
N_DUMMY = ACC_R - N_NODES
ROW_BLK = 2048

_mesh = plsc.VectorSubcoreMesh(
    core_axis_name="c", subcore_axis_name="s", num_cores=NC, num_subcores=NS)
_sc_params = pltpu.CompilerParams(needs_layout_passes=False)


@functools.partial(
    pl.kernel,
    out_type=(
        jax.ShapeDtypeStruct((NC * NS, HIST_W), jnp.float32),
        jax.ShapeDtypeStruct((E_PAD,), jnp.int32),
        jax.ShapeDtypeStruct((E_PAD,), jnp.int32),
        jax.ShapeDtypeStruct((E_PAD,), jnp.int32),
        jax.ShapeDtypeStruct((E_PAD,), jnp.int32),
    ),
    mesh=_mesh,
    scratch_types=[
        pltpu.VMEM((HIST_W,), jnp.float32),
        pltpu.VMEM((W_EDGES,), jnp.int32),
        pltpu.VMEM((W_EDGES,), jnp.int32),
        pltpu.VMEM((W_EDGES,), jnp.int32),
        pltpu.VMEM((W_EDGES,), jnp.int32),
        pltpu.VMEM((W_EDGES,), jnp.int32),
        pltpu.VMEM((W_EDGES,), jnp.int32),
    ],
    compiler_params=_sc_params,
)
def _prep_kernel(row_hbm, col_hbm,
                 hist_hbm, r0_hbm, c0_hbm, r1_hbm, c1_hbm,
                 hist_v, row_v, col_v, r0_v, c0_v, r1_v, c1_v):
    c = lax.axis_index("c")
    s = lax.axis_index("s")
    w = c * NS + s
    base = w * W_EDGES
    pltpu.sync_copy(row_hbm.at[pl.ds(base, W_EDGES)], row_v)
    pltpu.sync_copy(col_hbm.at[pl.ds(base, W_EDGES)], col_v)

    def zero_body(i, carry):
        hist_v[pl.ds(i * LANES, LANES)] = jnp.zeros((LANES,), jnp.float32)
        return carry

    lax.fori_loop(0, HIST_W // LANES, zero_body, 0)

    ones = jnp.ones((LANES,), jnp.float32)
    zeros_i = jnp.zeros((LANES,), jnp.int32)

    def step(i, carry):
        sl = pl.ds(i * LANES, LANES)
        r = row_v[sl]
        cl = col_v[sl]
        plsc.addupdate_scatter(hist_v, [cl], ones)
        dm = jnp.full((LANES,), N_NODES, jnp.int32) + lax.rem(i, N_DUMMY)
        in0 = r < HALF
        r0_v[sl] = jnp.where(in0, r, zeros_i)
        c0_v[sl] = jnp.where(in0, cl, dm)
        r1_v[sl] = jnp.where(in0, zeros_i, r - HALF)
        c1_v[sl] = jnp.where(in0, dm, cl)
        return carry

    lax.fori_loop(0, W_EDGES // LANES, step, 0)
    pltpu.sync_copy(hist_v, hist_hbm.at[w])
    pltpu.sync_copy(r0_v, r0_hbm.at[pl.ds(base, W_EDGES)])
    pltpu.sync_copy(c0_v, c0_hbm.at[pl.ds(base, W_EDGES)])
    pltpu.sync_copy(r1_v, r1_hbm.at[pl.ds(base, W_EDGES)])
    pltpu.sync_copy(c1_v, c1_hbm.at[pl.ds(base, W_EDGES)])


def _mm_body(x_ref, w_ref, hist_ref, hs0_ref, hs1_ref, dinv_ref):
    h = jnp.dot(x_ref[...], w_ref[...], preferred_element_type=jnp.float32)
    deg = jnp.sum(hist_ref[...], axis=0) + 1.0
    dinv = lax.rsqrt(deg)[:, None]
    hs = h * dinv
    hs0_ref[...] = hs[:, :IN_CH]
    hs1_ref[...] = hs[:, IN_CH:]
    dinv_ref[...] = dinv


def _mm_call(x, W, hist):
    grid = (HIST_W // ROW_BLK,)
    return pl.pallas_call(
        _mm_body,
        grid=grid,
        in_specs=[
            pl.BlockSpec((ROW_BLK, IN_CH), lambda i: (i, 0)),
            pl.BlockSpec((IN_CH, HID), lambda i: (0, 0)),
            pl.BlockSpec((NC * NS, ROW_BLK), lambda i: (0, i)),
        ],
        out_specs=[
            pl.BlockSpec((ROW_BLK, IN_CH), lambda i: (i, 0)),
            pl.BlockSpec((ROW_BLK, IN_CH), lambda i: (i, 0)),
            pl.BlockSpec((ROW_BLK, 1), lambda i: (i, 0)),
        ],
        out_shape=[
            jax.ShapeDtypeStruct((N_NODES, IN_CH), jnp.float32),
            jax.ShapeDtypeStruct((N_NODES, IN_CH), jnp.float32),
            jax.ShapeDtypeStruct((N_NODES, 1), jnp.float32),
        ],
    )(x, W, hist)


@functools.partial(
    pl.kernel,
    out_type=(
        jax.ShapeDtypeStruct((ACC_R, IN_CH), jnp.float32),
        jax.ShapeDtypeStruct((ACC_R, IN_CH), jnp.float32),
    ),
    mesh=_mesh,
    scratch_types=[
        pltpu.VMEM((4, SUPER, CHUNK), jnp.int32),
        pltpu.VMEM((2, CHUNK, IN_CH), jnp.float32),
        pltpu.VMEM_SHARED((HALF, IN_CH), jnp.float32),
        pltpu.VMEM_SHARED((ACC_R, IN_CH), jnp.float32),
        pltpu.SemaphoreType.DMA,
        pltpu.SemaphoreType.DMA,
        pltpu.SemaphoreType.DMA,
        pltpu.SemaphoreType.DMA,
    ],
    compiler_params=_sc_params,
)
def _scatter_kernel(hs0_hbm, hs1_hbm, z_hbm,
                    r0_hbm, c0_hbm, r1_hbm, c1_hbm,
                    out0_hbm, out1_hbm,
                    idx_v, bufs_v, tbl, acc,
                    sem_g, sem_sa, sem_sb, sem_i):
    c = lax.axis_index("c")
    s = lax.axis_index("s")
    idx_sets = ((idx_v.at[0], idx_v.at[1]), (idx_v.at[2], idx_v.at[3]))
    buf_a = bufs_v.at[0]
    buf_b = bufs_v.at[1]

    def one_pass(hs_hbm, half, ridx_hbm, cidx_hbm, first):
        @pl.when(s < NS - 1)
        def _():
            pltpu.sync_copy(hs_hbm.at[pl.ds(half * HALF + s * TSLAB, TSLAB)],
                            tbl.at[pl.ds(s * TSLAB, TSLAB)])

        @pl.when(s == NS - 1)
        def _():
            last = HALF - (NS - 1) * TSLAB
            pltpu.sync_copy(
                hs_hbm.at[pl.ds(half * HALF + (NS - 1) * TSLAB, last)],
                tbl.at[pl.ds((NS - 1) * TSLAB, last)])

        if first:
            pltpu.sync_copy(z_hbm, acc.at[pl.ds(s * SLAB, SLAB)])
        plsc.subcore_barrier()

        def fire_idx(st, g):
            rv, cv = idx_sets[st]
            pltpu.async_copy(ridx_hbm.at[s, g], rv, sem_i)
            pltpu.async_copy(cidx_hbm.at[s, g], cv, sem_i)

        def wait_idx(st, g):
            rv, cv = idx_sets[st]
            pltpu.make_async_copy(ridx_hbm.at[s, g], rv, sem_i).wait()
            pltpu.make_async_copy(cidx_hbm.at[s, g], cv, sem_i).wait()

        def super_pipe(st):
            rv, cv = idx_sets[st]

            def fire_g(buf, j):
                pltpu.async_copy(tbl.at[rv.at[j]], buf, sem_g)

            def wait_g(buf, j):
                pltpu.make_async_copy(tbl.at[rv.at[j]], buf, sem_g).wait()

            def fire_s(buf, j, sem):
                pltpu.async_copy(buf, acc.at[cv.at[j]], sem, add=True)

            def wait_s(buf, j, sem):
                pltpu.make_async_copy(buf, acc.at[cv.at[j]], sem).wait()

            fire_g(buf_a, 0)
            for p in range(SUPER // 2):
                q_a, q_b = 2 * p, 2 * p + 1
                wait_g(buf_a, q_a)
                fire_s(buf_a, q_a, sem_sa)
                if p >= 1:
                    wait_s(buf_b, q_a - 1, sem_sb)
                fire_g(buf_b, q_b)
                wait_g(buf_b, q_b)
                fire_s(buf_b, q_b, sem_sb)
                wait_s(buf_a, q_a, sem_sa)
                if p <= SUPER // 2 - 2:
                    fire_g(buf_a, q_b + 1)
            wait_s(buf_b, SUPER - 1, sem_sb)

        fire_idx(0, 0)

        def super_body(t, carry):
            g0 = 2 * t
            wait_idx(0, g0)
            fire_idx(1, g0 + 1)
            super_pipe(0)
            wait_idx(1, g0 + 1)

            @pl.when(t < N_SUPER // 2 - 1)
            def _():
                fire_idx(0, g0 + 2)

            super_pipe(1)
            return carry

        lax.fori_loop(0, N_SUPER // 2, super_body, 0)
        plsc.subcore_barrier()

    def run(hs_hbm, out_hbm):
        one_pass(hs_hbm, 0, r0_hbm, c0_hbm, True)
        one_pass(hs_hbm, 1, r1_hbm, c1_hbm, False)
        pltpu.sync_copy(acc.at[pl.ds(s * SLAB, SLAB)],
                        out_hbm.at[pl.ds(s * SLAB, SLAB)])

    @pl.when(c == 0)
    def _():
        run(hs0_hbm, out0_hbm)

    @pl.when(c == 1)
    def _():
        run(hs1_hbm, out1_hbm)


def _ep_body(acc0_ref, acc1_ref, hs0_ref, hs1_ref, dinv_ref, b_ref, a_ref,
             out_ref):
    m0 = acc0_ref[...] + hs0_ref[...]
    m1 = acc1_ref[...] + hs1_ref[...]
    m = jnp.concatenate([m0, m1], axis=1)
    pre = dinv_ref[...] * m + b_ref[...]
    p = jnp.where(pre > 0, pre, a_ref[...] * pre)
    nrm = jnp.sqrt(jnp.sum(p * p, axis=1, keepdims=True))
    out_ref[...] = p / jnp.maximum(nrm, 1e-12)


def _ep_call(acc0, acc1, hs0, hs1, dinv, b2, a2):
    grid = (HIST_W // ROW_BLK,)
    return pl.pallas_call(
        _ep_body,
        grid=grid,
        in_specs=[
            pl.BlockSpec((ROW_BLK, IN_CH), lambda i: (i, 0)),
            pl.BlockSpec((ROW_BLK, IN_CH), lambda i: (i, 0)),
            pl.BlockSpec((ROW_BLK, IN_CH), lambda i: (i, 0)),
            pl.BlockSpec((ROW_BLK, IN_CH), lambda i: (i, 0)),
            pl.BlockSpec((ROW_BLK, 1), lambda i: (i, 0)),
            pl.BlockSpec((1, HID), lambda i: (0, 0)),
            pl.BlockSpec((1, HID), lambda i: (0, 0)),
        ],
        out_specs=pl.BlockSpec((ROW_BLK, HID), lambda i: (i, 0)),
        out_shape=jax.ShapeDtypeStruct((N_NODES, HID), jnp.float32),
    )(acc0, acc1, hs0, hs1, dinv, b2, a2)


def kernel(x, edge_index, W, b, alpha):
    row = edge_index[0]
    col = edge_index[1]
    pad = E_PAD - N_EDGES
    row_p = jnp.concatenate([row, jnp.zeros((pad,), jnp.int32)])
    col_p = jnp.concatenate([col, jnp.full((pad,), N_NODES, jnp.int32)])

    hist, r0, c0, r1, c1 = _prep_kernel(row_p, col_p)
    hs0, hs1, dinv = _mm_call(x, W, hist)
    z = jnp.zeros((SLAB, IN_CH), jnp.float32)
    idx3 = [a.reshape(NS, N_SUPER, SUPER, CHUNK)
            for a in (r0, c0, r1, c1)]
    acc0, acc1 = _scatter_kernel(hs0, hs1, z, *idx3)
    out = _ep_call(acc0, acc1, hs0, hs1, dinv,
                   b.reshape(1, HID), alpha.reshape(1, HID))
    return out


# speedup vs baseline: 1.5052x; 1.5052x over previous
import functools

import jax
import jax.numpy as jnp
from jax import lax
from jax.experimental import pallas as pl
from jax.experimental.pallas import tpu as pltpu
from jax.experimental.pallas import tpu_sc as plsc

N_NODES = 10000
N_EDGES = 320000
IN_CH = 128
HID = 256

NC = 2
NS = 16
LANES = 16
NW = NC * NS

E_PAD = 320512
W_EDGES = E_PAD // NW
QP = 4
QSIZE = 2504
CHUNK = 80
SUPER = 8
SEG = SUPER * CHUNK
CAP_SUP = 16
LCAP = CAP_SUP * SEG
HIST_W = 10240
ACC_R = 10112
SLAB = ACC_R // NS
TSLAB = 160
N_DUMMY = ACC_R - N_NODES
ROW_BLK = 2048

_mesh = plsc.VectorSubcoreMesh(
    core_axis_name="c", subcore_axis_name="s", num_cores=NC, num_subcores=NS)
_sc_params = pltpu.CompilerParams(needs_layout_passes=False)


@functools.partial(
    pl.kernel,
    out_type=(
        jax.ShapeDtypeStruct((NW, HIST_W), jnp.float32),
        jax.ShapeDtypeStruct((NW, QP * LCAP), jnp.int32),
        jax.ShapeDtypeStruct((NW, QP * LCAP), jnp.int32),
        jax.ShapeDtypeStruct((NW, LANES), jnp.int32),
    ),
    mesh=_mesh,
    scratch_types=[
        pltpu.VMEM((HIST_W,), jnp.float32),
        pltpu.VMEM((W_EDGES,), jnp.int32),
        pltpu.VMEM((W_EDGES,), jnp.int32),
        pltpu.VMEM((QP * LCAP,), jnp.int32),
        pltpu.VMEM((QP * LCAP,), jnp.int32),
        pltpu.VMEM((LANES,), jnp.int32),
    ],
    compiler_params=_sc_params,
)
def _prep_kernel(row_hbm, col_hbm,
                 hist_hbm, rl_hbm, cl_hbm, cnt_hbm,
                 hist_v, row_v, col_v, rl_v, cl_v, cnt_v):
    c = lax.axis_index("c")
    s = lax.axis_index("s")
    w = c * NS + s
    base = w * W_EDGES
    pltpu.sync_copy(row_hbm.at[pl.ds(base, W_EDGES)], row_v)
    pltpu.sync_copy(col_hbm.at[pl.ds(base, W_EDGES)], col_v)

    zf = jnp.zeros((LANES,), jnp.float32)
    zi = jnp.zeros((LANES,), jnp.int32)
    ones = jnp.ones((LANES,), jnp.float32)

    def zero_body(i, carry):
        hist_v[pl.ds(i * LANES, LANES)] = zf
        return carry

    lax.fori_loop(0, HIST_W // LANES, zero_body, 0)

    def fill_body(i, carry):
        sl = pl.ds(i * LANES, LANES)
        rl_v[sl] = zi
        cl_v[sl] = jnp.full((LANES,), N_NODES, jnp.int32) + lax.rem(
            i, N_DUMMY)
        return carry

    lax.fori_loop(0, QP * LCAP // LANES, fill_body, 0)

    def step(i, cnts):
        sl = pl.ds(i * LANES, LANES)
        r = row_v[sl]
        cl = col_v[sl]
        plsc.addupdate_scatter(hist_v, [cl], ones)
        q = r // QSIZE
        new = []
        for p in range(QP):
            m = q == p
            rloc = r - p * QSIZE
            off = p * LCAP + cnts[p]
            plsc.store_compressed(rl_v.at[pl.ds(off, LANES)], rloc, mask=m)
            plsc.store_compressed(cl_v.at[pl.ds(off, LANES)], cl, mask=m)
            new.append(cnts[p] + jnp.sum(m.astype(jnp.int32)))
        return tuple(new)

    z32 = jnp.zeros((), jnp.int32)
    cnts = lax.fori_loop(0, W_EDGES // LANES, step, (z32, z32, z32, z32))

    lane = lax.iota(jnp.int32, LANES)
    cvec = zi
    for p in range(QP):
        ns_p = (cnts[p] + (SEG - 1)) // SEG
        cvec = jnp.where(lane == p, jnp.full((LANES,), 0, jnp.int32) + ns_p,
                         cvec)
    cnt_v[pl.ds(0, LANES)] = cvec

    pltpu.sync_copy(hist_v, hist_hbm.at[w])
    pltpu.sync_copy(rl_v, rl_hbm.at[w])
    pltpu.sync_copy(cl_v, cl_hbm.at[w])
    pltpu.sync_copy(cnt_v, cnt_hbm.at[w])


def _mm_body(x_ref, w_ref, hist_ref, hs0_ref, hs1_ref, dinv_ref):
    h = jnp.dot(x_ref[...], w_ref[...], preferred_element_type=jnp.float32)
    deg = jnp.sum(hist_ref[...], axis=0) + 1.0
    dinv = lax.rsqrt(deg)[:, None]
    hs = h * dinv
    hs0_ref[...] = hs[:, :IN_CH]
    hs1_ref[...] = hs[:, IN_CH:]
    dinv_ref[...] = dinv


def _mm_call(x, W, hist):
    grid = (HIST_W // ROW_BLK,)
    return pl.pallas_call(
        _mm_body,
        grid=grid,
        in_specs=[
            pl.BlockSpec((ROW_BLK, IN_CH), lambda i: (i, 0)),
            pl.BlockSpec((IN_CH, HID), lambda i: (0, 0)),
            pl.BlockSpec((NW, ROW_BLK), lambda i: (0, i)),
        ],
        out_specs=[
            pl.BlockSpec((ROW_BLK, IN_CH), lambda i: (i, 0)),
            pl.BlockSpec((ROW_BLK, IN_CH), lambda i: (i, 0)),
            pl.BlockSpec((ROW_BLK, 1), lambda i: (i, 0)),
        ],
        out_shape=[
            jax.ShapeDtypeStruct((N_NODES, IN_CH), jnp.float32),
            jax.ShapeDtypeStruct((N_NODES, IN_CH), jnp.float32),
            jax.ShapeDtypeStruct((N_NODES, 1), jnp.float32),
        ],
    )(x, W, hist)


@functools.partial(
    pl.kernel,
    out_type=(
        jax.ShapeDtypeStruct((ACC_R, IN_CH), jnp.float32),
        jax.ShapeDtypeStruct((ACC_R, IN_CH), jnp.float32),
    ),
    mesh=_mesh,
    scratch_types=[
        pltpu.VMEM((2, SUPER, CHUNK), jnp.int32),
        pltpu.VMEM((2, CHUNK, IN_CH), jnp.float32),
        pltpu.VMEM((2, LANES), jnp.int32),
        pltpu.VMEM_SHARED((QSIZE, IN_CH), jnp.float32),
        pltpu.VMEM_SHARED((ACC_R, IN_CH), jnp.float32),
        pltpu.SemaphoreType.DMA,
        pltpu.SemaphoreType.DMA,
        pltpu.SemaphoreType.DMA,
    ],
    compiler_params=_sc_params,
)
def _scatter_kernel(hs0_hbm, hs1_hbm, z_hbm, rl_hbm, cl_hbm, cnt_hbm,
                    out0_hbm, out1_hbm,
                    idx_v, bufs_v, cnt_v, tbl, acc,
                    sem_g, sem_sa, sem_sb):
    c = lax.axis_index("c")
    s = lax.axis_index("s")
    buf_a = bufs_v.at[0]
    buf_b = bufs_v.at[1]
    rv = idx_v.at[0]
    cv = idx_v.at[1]

    def run(hs_hbm, out_hbm):
        pltpu.sync_copy(cnt_hbm.at[s], cnt_v)
        pltpu.sync_copy(z_hbm, acc.at[pl.ds(s * SLAB, SLAB)])

        def super_pipe():
            def fire_g(buf, j):
                pltpu.async_copy(tbl.at[rv.at[j]], buf, sem_g)

            def wait_g(buf, j):
                pltpu.make_async_copy(tbl.at[rv.at[j]], buf, sem_g).wait()

            def fire_s(buf, j, sem):
                pltpu.async_copy(buf, acc.at[cv.at[j]], sem, add=True)

            def wait_s(buf, j, sem):
                pltpu.make_async_copy(buf, acc.at[cv.at[j]], sem).wait()

            fire_g(buf_a, 0)
            for p in range(SUPER // 2):
                q_a, q_b = 2 * p, 2 * p + 1
                wait_g(buf_a, q_a)
                fire_s(buf_a, q_a, sem_sa)
                if p >= 1:
                    wait_s(buf_b, q_a - 1, sem_sb)
                fire_g(buf_b, q_b)
                wait_g(buf_b, q_b)
                fire_s(buf_b, q_b, sem_sb)
                wait_s(buf_a, q_a, sem_sa)
                if p <= SUPER // 2 - 2:
                    fire_g(buf_a, q_b + 1)
            wait_s(buf_b, SUPER - 1, sem_sb)

        for p in range(QP):
            tlen = min(QSIZE, N_NODES - p * QSIZE)

            @pl.when(s < NS - 1)
            def _():
                pltpu.sync_copy(
                    hs_hbm.at[pl.ds(p * QSIZE + s * TSLAB, TSLAB)],
                    tbl.at[pl.ds(s * TSLAB, TSLAB)])

            @pl.when(s == NS - 1)
            def _():
                last = tlen - (NS - 1) * TSLAB
                pltpu.sync_copy(
                    hs_hbm.at[pl.ds(p * QSIZE + (NS - 1) * TSLAB, last)],
                    tbl.at[pl.ds((NS - 1) * TSLAB, last)])

            plsc.subcore_barrier()

            for i in range(2):
                wi = 2 * s + i
                n_sup = cnt_v[i, pl.ds(0, LANES)][p]

                def sup_body(g, carry):
                    pltpu.sync_copy(rl_hbm.at[wi, p, g], rv)
                    pltpu.sync_copy(cl_hbm.at[wi, p, g], cv)
                    super_pipe()
                    return carry

                lax.fori_loop(0, n_sup, sup_body, 0)

            plsc.subcore_barrier()

        pltpu.sync_copy(acc.at[pl.ds(s * SLAB, SLAB)],
                        out_hbm.at[pl.ds(s * SLAB, SLAB)])

    @pl.when(c == 0)
    def _():
        run(hs0_hbm, out0_hbm)

    @pl.when(c == 1)
    def _():
        run(hs1_hbm, out1_hbm)


def _ep_body(acc0_ref, acc1_ref, hs0_ref, hs1_ref, dinv_ref, b_ref, a_ref,
             out_ref):
    m0 = acc0_ref[...] + hs0_ref[...]
    m1 = acc1_ref[...] + hs1_ref[...]
    m = jnp.concatenate([m0, m1], axis=1)
    pre = dinv_ref[...] * m + b_ref[...]
    p = jnp.where(pre > 0, pre, a_ref[...] * pre)
    nrm = jnp.sqrt(jnp.sum(p * p, axis=1, keepdims=True))
    out_ref[...] = p / jnp.maximum(nrm, 1e-12)


def _ep_call(acc0, acc1, hs0, hs1, dinv, b2, a2):
    grid = (HIST_W // ROW_BLK,)
    return pl.pallas_call(
        _ep_body,
        grid=grid,
        in_specs=[
            pl.BlockSpec((ROW_BLK, IN_CH), lambda i: (i, 0)),
            pl.BlockSpec((ROW_BLK, IN_CH), lambda i: (i, 0)),
            pl.BlockSpec((ROW_BLK, IN_CH), lambda i: (i, 0)),
            pl.BlockSpec((ROW_BLK, IN_CH), lambda i: (i, 0)),
            pl.BlockSpec((ROW_BLK, 1), lambda i: (i, 0)),
            pl.BlockSpec((1, HID), lambda i: (0, 0)),
            pl.BlockSpec((1, HID), lambda i: (0, 0)),
        ],
        out_specs=pl.BlockSpec((ROW_BLK, HID), lambda i: (i, 0)),
        out_shape=jax.ShapeDtypeStruct((N_NODES, HID), jnp.float32),
    )(acc0, acc1, hs0, hs1, dinv, b2, a2)


def kernel(x, edge_index, W, b, alpha):
    row = edge_index[0]
    col = edge_index[1]
    pad = E_PAD - N_EDGES
    row_p = jnp.concatenate([row, jnp.zeros((pad,), jnp.int32)])
    col_p = jnp.concatenate([col, jnp.full((pad,), N_NODES, jnp.int32)])

    hist, rl, cl, cnt = _prep_kernel(row_p, col_p)
    hs0, hs1, dinv = _mm_call(x, W, hist)
    z = jnp.zeros((SLAB, IN_CH), jnp.float32)
    rl4 = rl.reshape(NW, QP, CAP_SUP, SUPER, CHUNK)
    cl4 = cl.reshape(NW, QP, CAP_SUP, SUPER, CHUNK)
    cnt3 = cnt.reshape(NS, 2, LANES)
    acc0, acc1 = _scatter_kernel(hs0, hs1, z, rl4, cl4, cnt3)
    out = _ep_call(acc0, acc1, hs0, hs1, dinv,
                   b.reshape(1, HID), alpha.reshape(1, HID))
    return out
